# Initial kernel scaffold; baseline (speedup 1.0000x reference)
import jax
import jax.numpy as jnp
from jax.experimental import pallas as pl


def kernel(x, edge_index, batch, W1, b1, g1, bt1, W2, b2, W3, b3, W4, b4, W5, b5, W6, b6, W7, b7):
    raise NotImplementedError("write your pallas kernel here")


# speedup vs baseline: 6.5564x; 6.5564x over previous
import functools

import jax
import jax.numpy as jnp
from jax import lax
from jax.experimental import pallas as pl
from jax.experimental.pallas import tpu as pltpu
from jax.experimental.pallas import tpu_sc as plsc

N = 10000
E = 320000
D = 128
H = 64
OUT = 10
G = 64

NC = 2
NS = 16
NW = NC * NS
CHUNK = 128
NCHUNKS = -(-E // (NW * CHUNK))
EPW = NCHUNKS * CHUNK
EPAD = EPW * NW
NROWS = 10112
RPT = NROWS // NS


def _sc_agg_body(table, srcs, dsts, zinit, out, src_v, dst_v, rows_v, sem):
    c = lax.axis_index("c")
    s = lax.axis_index("s")
    wid = c * NS + s
    pltpu.sync_copy(srcs.at[wid], src_v)
    pltpu.sync_copy(dsts.at[wid], dst_v)
    pltpu.sync_copy(zinit.at[pl.ds(s * RPT, RPT)],
                    out.at[c, pl.ds(s * RPT, RPT)])
    plsc.subcore_barrier()

    def step(j, carry):
        pltpu.async_copy(table.at[src_v.at[j]], rows_v, sem).wait()
        pltpu.sync_copy(rows_v, out.at[c].at[dst_v.at[j]], add=True)
        return carry

    lax.fori_loop(0, NCHUNKS, step, 0, unroll=False)
    return


def _sc_agg_body_spmem(table, srcs, dsts, zinit, out, src_v, dst_v, rows_v,
                       acc, sem):
    c = lax.axis_index("c")
    s = lax.axis_index("s")
    wid = c * NS + s
    pltpu.sync_copy(srcs.at[wid], src_v)
    pltpu.sync_copy(dsts.at[wid], dst_v)
    pltpu.sync_copy(zinit.at[pl.ds(s * RPT, RPT)], acc.at[pl.ds(s * RPT, RPT)])
    plsc.subcore_barrier()

    def step(j, carry):
        pltpu.async_copy(table.at[src_v.at[j]], rows_v, sem).wait()
        pltpu.sync_copy(rows_v, acc.at[dst_v.at[j]], add=True)
        return carry

    lax.fori_loop(0, NCHUNKS, step, 0, unroll=False)
    plsc.subcore_barrier()
    pltpu.sync_copy(acc.at[pl.ds(s * RPT, RPT)],
                    out.at[c, pl.ds(s * RPT, RPT)])
    return


def _sc_agg(table, srcs, dsts, zinit):
    mesh = plsc.VectorSubcoreMesh(core_axis_name="c", subcore_axis_name="s")
    f = pl.kernel(
        _sc_agg_body_spmem,
        out_type=jax.ShapeDtypeStruct((NC, NROWS, H), jnp.float32),
        mesh=mesh,
        scratch_types=[
            pltpu.VMEM((NCHUNKS, CHUNK), jnp.int32),
            pltpu.VMEM((NCHUNKS, CHUNK), jnp.int32),
            pltpu.VMEM((CHUNK, H), jnp.float32),
            pltpu.VMEM_SHARED((NROWS, H), jnp.float32),
            pltpu.SemaphoreType.DMA,
        ],
        compiler_params=pltpu.CompilerParams(use_tc_tiling_on_sc=False),
    )
    return f(table, srcs, dsts, zinit)


def _tc_matmul_body(x_ref, w_ref, o_ref):
    o_ref[...] = jnp.dot(x_ref[...], w_ref[...],
                         preferred_element_type=jnp.float32)


def _tc_matmul(x, w):
    return pl.pallas_call(
        _tc_matmul_body,
        out_shape=jax.ShapeDtypeStruct((x.shape[0], w.shape[1]), jnp.float32),
    )(x, w)


def _elu(v):
    return jnp.where(v > 0, v, jnp.exp(v) - 1.0)


def _stage2_body(xw_ref, p_ref, b1_ref, g1_ref, bt1_ref, w2_ref, b2_ref,
                 w3_ref, o_ref):
    p = p_ref[...]
    a = xw_ref[...] + p[0, :N, :] + p[1, :N, :] + b1_ref[...]
    mu = jnp.mean(a, axis=0, keepdims=True)
    var = jnp.mean((a - mu) ** 2, axis=0, keepdims=True)
    h = (a - mu) * lax.rsqrt(var + 1e-5) * g1_ref[...] + bt1_ref[...]
    h = _elu(h)
    h = jnp.dot(h, w2_ref[...], preferred_element_type=jnp.float32)
    h = _elu(h + b2_ref[...])
    o_ref[...] = jnp.dot(h, w3_ref[...], preferred_element_type=jnp.float32)


def _stage2(xw1, parts, b1, g1, bt1, w2, b2, w3):
    return pl.pallas_call(
        _stage2_body,
        out_shape=jax.ShapeDtypeStruct((N, H), jnp.float32),
    )(xw1, parts, b1, g1, bt1, w2, b2, w3)


def _stage3_body(hw_ref, p_ref, b3_ref, w4_ref, b4_ref, batch_ref, w5_ref,
                 b5_ref, w6_ref, b6_ref, w7_ref, b7_ref, o_ref):
    p = p_ref[...]
    a = hw_ref[...] + p[0, :N, :] + p[1, :N, :] + b3_ref[...]
    h = _elu(a)
    h = jnp.dot(h, w4_ref[...], preferred_element_type=jnp.float32)
    h = _elu(h + b4_ref[...])
    ohT = (batch_ref[...] ==
           lax.broadcasted_iota(jnp.int32, (G, N), 0)).astype(jnp.float32)
    g = jnp.dot(ohT, h, preferred_element_type=jnp.float32)
    g = _elu(jnp.dot(g, w5_ref[...], preferred_element_type=jnp.float32)
             + b5_ref[...])
    g = _elu(jnp.dot(g, w6_ref[...], preferred_element_type=jnp.float32)
             + b6_ref[...])
    logits = jnp.dot(g, w7_ref[...], preferred_element_type=jnp.float32)
    logits = logits + b7_ref[...]
    m = jnp.max(logits, axis=-1, keepdims=True)
    lse = jnp.log(jnp.sum(jnp.exp(logits - m), axis=-1, keepdims=True)) + m
    o_ref[...] = logits - lse


def _stage3(hw3, parts, b3, w4, b4, batch2d, w5, b5, w6, b6, w7, b7):
    return pl.pallas_call(
        _stage3_body,
        out_shape=jax.ShapeDtypeStruct((G, OUT), jnp.float32),
    )(hw3, parts, b3, w4, b4, batch2d, w5, b5, w6, b6, w7, b7)


def kernel(x, edge_index, batch, W1, b1, g1, bt1, W2, b2, W3, b3, W4, b4,
           W5, b5, W6, b6, W7, b7):
    src = edge_index[0]
    dst = edge_index[1]
    pad = EPAD - E
    src_p = jnp.concatenate([src, jnp.zeros((pad,), jnp.int32)])
    dst_p = jnp.concatenate([dst, jnp.full((pad,), N, jnp.int32)])
    srcs = src_p.reshape(NW, NCHUNKS, CHUNK)
    dsts = dst_p.reshape(NW, NCHUNKS, CHUNK)
    zinit = jnp.zeros((NROWS, H), jnp.float32)
    b1r = b1.reshape(1, H)
    g1r = g1.reshape(1, H)
    bt1r = bt1.reshape(1, H)
    b2r = b2.reshape(1, H)
    b3r = b3.reshape(1, H)
    b4r = b4.reshape(1, H)
    b5r = b5.reshape(1, H)
    b6r = b6.reshape(1, H // 2)
    b7r = b7.reshape(1, OUT)
    batch2d = batch.reshape(1, N)

    xw1 = _tc_matmul(x, W1)
    parts1 = _sc_agg(xw1, srcs, dsts, zinit)
    hw3 = _stage2(xw1, parts1, b1r, g1r, bt1r, W2, b2r, W3)
    parts2 = _sc_agg(hw3, srcs, dsts, zinit)
    return _stage3(hw3, parts2, b3r, W4, b4r, batch2d, W5, b5r, W6, b6r,
                   W7, b7r)
